# baseline (speedup 1.0000x reference)
import functools

import jax
import jax.numpy as jnp
from jax import lax
from jax.experimental import pallas as pl
from jax.experimental.pallas import tpu as pltpu
from jax.experimental.pallas import tpu_sc as plsc

H, E, K = 1024, 8, 2
N = 4096
BLK_R = 256
G = 40
P = G * BLK_R
RBLK = 256

NC, NS = 2, 16
NW = NC * NS
DROWS_W = P // NW
DCH = 16
TOK_W = N // NW
TCH = 16


def _router_body(x_ref, rw_ref, i1_ref, i2_ref, w1_ref, w2_ref, b_ref,
                 r1_ref, r2_ref, c_ref, base_ref):
    @pl.when(pl.program_id(0) == 0)
    def _init():
        base_ref[...] = jnp.zeros_like(base_ref)

    x = x_ref[...]
    logits = jnp.dot(x, rw_ref[...].T, preferred_element_type=jnp.float32)
    m = jnp.max(logits, axis=-1, keepdims=True)
    p = jnp.exp(logits - m)
    probs = p / jnp.sum(p, axis=-1, keepdims=True)
    iota = jax.lax.broadcasted_iota(jnp.int32, probs.shape, 1)
    m1 = jnp.max(probs, axis=-1, keepdims=True)
    i1 = jnp.min(jnp.where(probs == m1, iota, E), axis=-1, keepdims=True)
    probs2 = jnp.where(iota == i1, -jnp.inf, probs)
    m2 = jnp.max(probs2, axis=-1, keepdims=True)
    i2 = jnp.min(jnp.where(probs2 == m2, iota, E), axis=-1, keepdims=True)
    denom = m1 + m2 + 1e-6
    w1 = m1 / denom
    w2 = m2 / denom
    i1_ref[...] = i1
    i2_ref[...] = i2
    w1_ref[...] = w1
    w2_ref[...] = w2
    b_ref[...] = w1 * m1 + w2 * m2
    oh1 = (iota == i1).astype(jnp.float32)
    oh2 = (iota == i2).astype(jnp.float32)
    ri = jax.lax.broadcasted_iota(jnp.int32, (RBLK, RBLK), 0)
    ci = jax.lax.broadcasted_iota(jnp.int32, (RBLK, RBLK), 1)
    tri = (ri > ci).astype(jnp.float32)
    base = base_ref[...]
    r1 = base + jnp.dot(tri, oh1, preferred_element_type=jnp.float32)
    base = base + jnp.sum(oh1, axis=0, keepdims=True)
    r2 = base + jnp.dot(tri, oh2, preferred_element_type=jnp.float32)
    base = base + jnp.sum(oh2, axis=0, keepdims=True)
    base_ref[...] = base
    r1_ref[...] = jnp.sum(r1 * oh1, axis=-1, keepdims=True).astype(jnp.int32)
    r2_ref[...] = jnp.sum(r2 * oh2, axis=-1, keepdims=True).astype(jnp.int32)
    c_ref[...] = base


def _router(x, router_w):
    o = jax.ShapeDtypeStruct((N, 1), jnp.float32)
    oi = jax.ShapeDtypeStruct((N, 1), jnp.int32)
    spec = pl.BlockSpec((RBLK, 1), lambda i: (i, 0))
    return pl.pallas_call(
        _router_body,
        grid=(N // RBLK,),
        in_specs=[
            pl.BlockSpec((RBLK, H), lambda i: (i, 0)),
            pl.BlockSpec((E, H), lambda i: (0, 0)),
        ],
        out_specs=[spec, spec, spec, spec, spec, spec, spec,
                   pl.BlockSpec((1, E), lambda i: (0, 0))],
        out_shape=[oi, oi, o, o, o, oi, oi,
                   jax.ShapeDtypeStruct((1, E), jnp.float32)],
        scratch_shapes=[pltpu.VMEM((1, E), jnp.float32)],
    )(x, router_w)


_SC_MESH = plsc.VectorSubcoreMesh(core_axis_name="c", subcore_axis_name="s")


_RING = 6
_LOOK = 3


@functools.partial(
    pl.kernel,
    out_type=jax.ShapeDtypeStruct((P, H), jnp.float32),
    mesh=_SC_MESH,
    scratch_types=(
        [pltpu.VMEM((DCH,), jnp.int32) for _ in range(_RING)]
        + [pltpu.VMEM((DCH, H), jnp.float32) for _ in range(_RING)]
        + [pltpu.SemaphoreType.DMA for _ in range(2 * _RING)]
    ),
)
def _dispatch_gather(x_hbm, idx_hbm, out_hbm, *scr):
    idxs = scr[:_RING]
    rows = scr[_RING:2 * _RING]
    gsem = scr[2 * _RING:3 * _RING]
    ssem = scr[3 * _RING:4 * _RING]
    wid = lax.axis_index("s") * NC + lax.axis_index("c")
    base = wid * DROWS_W
    NCH = DROWS_W // DCH

    def start(c, b):
        pltpu.sync_copy(idx_hbm.at[pl.ds(base + c * DCH, DCH)], idxs[b])
        return pltpu.async_copy(x_hbm.at[idxs[b]], rows[b], gsem[b])

    gaths = [None] * _RING
    stores = [None] * _RING
    for j in range(min(_LOOK, NCH)):
        gaths[j % _RING] = start(j, j % _RING)
    for c in range(NCH):
        b = c % _RING
        n = c + _LOOK
        if n < NCH:
            nb = n % _RING
            if stores[nb] is not None:
                stores[nb].wait()
                stores[nb] = None
            gaths[nb] = start(n, nb)
        gaths[b].wait()
        stores[b] = pltpu.async_copy(
            rows[b], out_hbm.at[pl.ds(base + c * DCH, DCH)], ssem[b])
    for st in stores:
        if st is not None:
            st.wait()


@functools.partial(
    pl.kernel,
    out_type=jax.ShapeDtypeStruct((N, H), jnp.float32),
    mesh=_SC_MESH,
    scratch_types=[
        pltpu.VMEM((TCH,), jnp.int32),
        pltpu.VMEM((TCH,), jnp.int32),
        pltpu.VMEM((TCH,), jnp.int32),
        pltpu.VMEM((TCH,), jnp.int32),
        pltpu.VMEM((TCH, H), jnp.float32),
        pltpu.VMEM((TCH, H), jnp.float32),
        pltpu.VMEM((TCH, H), jnp.float32),
        pltpu.VMEM((TCH, H), jnp.float32),
        pltpu.SemaphoreType.DMA,
        pltpu.SemaphoreType.DMA,
        pltpu.SemaphoreType.DMA,
        pltpu.SemaphoreType.DMA,
        pltpu.SemaphoreType.DMA,
        pltpu.SemaphoreType.DMA,
    ],
)
def _combine(y_hbm, p1_hbm, p2_hbm, out_hbm, p1a, p2a, p1b, p2b,
             a0, b0, a1, b1, ga0, gb0, ga1, gb1, ss0, ss1):
    wid = lax.axis_index("s") * NC + lax.axis_index("c")
    base = wid * TOK_W
    NCH = TOK_W // TCH
    p1s, p2s = (p1a, p1b), (p2a, p2b)
    avs, bvs = (a0, a1), (b0, b1)
    gsa, gsb, ssem = (ga0, ga1), (gb0, gb1), (ss0, ss1)

    def start(c, b):
        off = base + c * TCH
        pltpu.sync_copy(p1_hbm.at[pl.ds(off, TCH)], p1s[b])
        pltpu.sync_copy(p2_hbm.at[pl.ds(off, TCH)], p2s[b])
        return (pltpu.async_copy(y_hbm.at[p1s[b]], avs[b], gsa[b]),
                pltpu.async_copy(y_hbm.at[p2s[b]], bvs[b], gsb[b]))

    stores = [None, None]
    gn = start(0, 0)
    for c in range(NCH):
        b = c & 1
        gcur = gn
        if c + 1 < NCH:
            if stores[b ^ 1] is not None:
                stores[b ^ 1].wait()
                stores[b ^ 1] = None
            gn = start(c + 1, b ^ 1)
        gcur[0].wait()
        gcur[1].wait()
        a_v, b_v = avs[b], bvs[b]

        def add_row(r, _):
            for cc in range(H // 16):
                a_v[r, pl.ds(cc * 16, 16)] += b_v[r, pl.ds(cc * 16, 16)]
            return 0

        lax.fori_loop(0, TCH, add_row, 0)
        stores[b] = pltpu.async_copy(
            a_v, out_hbm.at[pl.ds(base + c * TCH, TCH)], ssem[b])
    for st in stores:
        if st is not None:
            st.wait()


def _gmm_body(eb_ref, x_ref, w_ref, wa_ref, bz_ref, y_ref):
    y = jnp.dot(x_ref[...].astype(jnp.bfloat16), w_ref[0],
                preferred_element_type=jnp.float32)
    y_ref[...] = y * wa_ref[...] + bz_ref[...]


def _gmm(eb, x_sorted, ws, wa_pad, bz_pad):
    grid_spec = pltpu.PrefetchScalarGridSpec(
        num_scalar_prefetch=1,
        grid=(G,),
        in_specs=[
            pl.BlockSpec((BLK_R, H), lambda t, eb: (t, 0)),
            pl.BlockSpec((1, H, H), lambda t, eb: (eb[t], 0, 0)),
            pl.BlockSpec((BLK_R, 1), lambda t, eb: (t, 0)),
            pl.BlockSpec((BLK_R, 1), lambda t, eb: (t, 0)),
        ],
        out_specs=pl.BlockSpec((BLK_R, H), lambda t, eb: (t, 0)),
    )
    return pl.pallas_call(
        _gmm_body,
        grid_spec=grid_spec,
        out_shape=jax.ShapeDtypeStruct((P, H), jnp.float32),
    )(eb, x_sorted, ws, wa_pad, bz_pad)


@jax.jit
def _moe(x, router_w, ws_bf16):
    i1, i2, w1, w2, bias, r1, r2, cnt = _router(x, router_w)
    i1, i2 = i1[:, 0], i2[:, 0]
    r1, r2 = r1[:, 0], r2[:, 0]
    counts = cnt.reshape(E).astype(jnp.int32)
    nblk = -(-counts // BLK_R)
    blk_off = jnp.concatenate([jnp.zeros(1, jnp.int32),
                               jnp.cumsum(nblk)[:-1].astype(jnp.int32)])
    cum_nblk = jnp.cumsum(nblk).astype(jnp.int32)
    t_iota = jnp.arange(G, dtype=jnp.int32)
    eb = jnp.minimum(jnp.searchsorted(cum_nblk, t_iota, side='right')
                     .astype(jnp.int32), E - 1)
    p1 = blk_off[i1] * BLK_R + r1
    p2 = blk_off[i2] * BLK_R + r2
    tok = jnp.arange(N, dtype=jnp.int32)
    src_row = jnp.zeros(P, jnp.int32).at[p1].set(tok).at[p2].set(tok)
    wa_pad = (jnp.zeros(P, jnp.float32).at[p1].set(w1[:, 0])
              .at[p2].set(w2[:, 0]))[:, None]
    bz_pad = jnp.zeros(P, jnp.float32).at[p1].set(bias[:, 0])[:, None]
    x_sorted = _dispatch_gather(x, src_row)
    y = _gmm(eb, x_sorted, ws_bf16, wa_pad, bz_pad)
    out = _combine(y, p1, p2)
    return out


def kernel(hidden_states, router_w, expert_weights, expert_mapping):
    b, s, h = hidden_states.shape
    x = hidden_states.reshape(-1, h)
    out = _moe(x, router_w, expert_weights.astype(jnp.bfloat16))
    return out.reshape(b, s, h)


# speedup vs baseline: 1.2364x; 1.2364x over previous
import functools

import jax
import jax.numpy as jnp
from jax import lax
from jax.experimental import pallas as pl
from jax.experimental.pallas import tpu as pltpu
from jax.experimental.pallas import tpu_sc as plsc

H, E, K = 1024, 8, 2
N = 4096
BLK_R = 256
G = 40
P = G * BLK_R
RBLK = 256

NC, NS = 2, 16
NW = NC * NS
DROWS_W = P // NW
DCH = 16
TOK_W = N // NW
TCH = 16


def _router_body(x_ref, rw_ref, i1_ref, i2_ref, w1_ref, w2_ref, b_ref,
                 r1_ref, r2_ref, c_ref, base_ref):
    @pl.when(pl.program_id(0) == 0)
    def _init():
        base_ref[...] = jnp.zeros_like(base_ref)

    x = x_ref[...]
    logits = jnp.dot(x, rw_ref[...].T, preferred_element_type=jnp.float32)
    m = jnp.max(logits, axis=-1, keepdims=True)
    p = jnp.exp(logits - m)
    probs = p / jnp.sum(p, axis=-1, keepdims=True)
    iota = jax.lax.broadcasted_iota(jnp.int32, probs.shape, 1)
    m1 = jnp.max(probs, axis=-1, keepdims=True)
    i1 = jnp.min(jnp.where(probs == m1, iota, E), axis=-1, keepdims=True)
    probs2 = jnp.where(iota == i1, -jnp.inf, probs)
    m2 = jnp.max(probs2, axis=-1, keepdims=True)
    i2 = jnp.min(jnp.where(probs2 == m2, iota, E), axis=-1, keepdims=True)
    denom = m1 + m2 + 1e-6
    w1 = m1 / denom
    w2 = m2 / denom
    i1_ref[...] = i1
    i2_ref[...] = i2
    w1_ref[...] = w1
    w2_ref[...] = w2
    b_ref[...] = w1 * m1 + w2 * m2
    oh1 = (iota == i1).astype(jnp.float32)
    oh2 = (iota == i2).astype(jnp.float32)
    ri = jax.lax.broadcasted_iota(jnp.int32, (RBLK, RBLK), 0)
    ci = jax.lax.broadcasted_iota(jnp.int32, (RBLK, RBLK), 1)
    tri = (ri > ci).astype(jnp.float32)
    base = base_ref[...]
    r1 = base + jnp.dot(tri, oh1, preferred_element_type=jnp.float32)
    base = base + jnp.sum(oh1, axis=0, keepdims=True)
    r2 = base + jnp.dot(tri, oh2, preferred_element_type=jnp.float32)
    base = base + jnp.sum(oh2, axis=0, keepdims=True)
    base_ref[...] = base
    r1_ref[...] = jnp.sum(r1 * oh1, axis=-1, keepdims=True).astype(jnp.int32)
    r2_ref[...] = jnp.sum(r2 * oh2, axis=-1, keepdims=True).astype(jnp.int32)
    c_ref[...] = base


def _router(x, router_w):
    o = jax.ShapeDtypeStruct((N, 1), jnp.float32)
    oi = jax.ShapeDtypeStruct((N, 1), jnp.int32)
    spec = pl.BlockSpec((RBLK, 1), lambda i: (i, 0))
    return pl.pallas_call(
        _router_body,
        grid=(N // RBLK,),
        in_specs=[
            pl.BlockSpec((RBLK, H), lambda i: (i, 0)),
            pl.BlockSpec((E, H), lambda i: (0, 0)),
        ],
        out_specs=[spec, spec, spec, spec, spec, spec, spec,
                   pl.BlockSpec((1, E), lambda i: (0, 0))],
        out_shape=[oi, oi, o, o, o, oi, oi,
                   jax.ShapeDtypeStruct((1, E), jnp.float32)],
        scratch_shapes=[pltpu.VMEM((1, E), jnp.float32)],
    )(x, router_w)


_SC_MESH = plsc.VectorSubcoreMesh(core_axis_name="c", subcore_axis_name="s")


_RING = 6
_LOOK = 3


@functools.partial(
    pl.kernel,
    out_type=jax.ShapeDtypeStruct((P, H), jnp.float32),
    mesh=_SC_MESH,
    scratch_types=(
        [pltpu.VMEM((DCH,), jnp.int32) for _ in range(_RING)]
        + [pltpu.VMEM((DCH, H), jnp.float32) for _ in range(_RING)]
        + [pltpu.SemaphoreType.DMA for _ in range(2 * _RING)]
    ),
)
def _dispatch_gather(x_hbm, idx_hbm, out_hbm, *scr):
    idxs = scr[:_RING]
    rows = scr[_RING:2 * _RING]
    gsem = scr[2 * _RING:3 * _RING]
    ssem = scr[3 * _RING:4 * _RING]
    wid = lax.axis_index("s") * NC + lax.axis_index("c")
    base = wid * DROWS_W
    NCH = DROWS_W // DCH

    def start(c, b):
        pltpu.sync_copy(idx_hbm.at[pl.ds(base + c * DCH, DCH)], idxs[b])
        return pltpu.async_copy(x_hbm.at[idxs[b]], rows[b], gsem[b])

    gaths = [None] * _RING
    stores = [None] * _RING
    for j in range(min(_LOOK, NCH)):
        gaths[j % _RING] = start(j, j % _RING)
    for c in range(NCH):
        b = c % _RING
        n = c + _LOOK
        if n < NCH:
            nb = n % _RING
            if stores[nb] is not None:
                stores[nb].wait()
                stores[nb] = None
            gaths[nb] = start(n, nb)
        gaths[b].wait()
        stores[b] = pltpu.async_copy(
            rows[b], out_hbm.at[pl.ds(base + c * DCH, DCH)], ssem[b])
    for st in stores:
        if st is not None:
            st.wait()


@functools.partial(
    pl.kernel,
    out_type=jax.ShapeDtypeStruct((N, H), jnp.float32),
    mesh=_SC_MESH,
    scratch_types=[
        pltpu.VMEM((TCH,), jnp.int32),
        pltpu.VMEM((TCH,), jnp.int32),
        pltpu.VMEM((TCH,), jnp.int32),
        pltpu.VMEM((TCH,), jnp.int32),
        pltpu.VMEM((TCH, H), jnp.float32),
        pltpu.VMEM((TCH, H), jnp.float32),
        pltpu.VMEM((TCH, H), jnp.float32),
        pltpu.VMEM((TCH, H), jnp.float32),
        pltpu.SemaphoreType.DMA,
        pltpu.SemaphoreType.DMA,
        pltpu.SemaphoreType.DMA,
        pltpu.SemaphoreType.DMA,
        pltpu.SemaphoreType.DMA,
        pltpu.SemaphoreType.DMA,
    ],
)
def _combine(y_hbm, p1_hbm, p2_hbm, out_hbm, p1a, p2a, p1b, p2b,
             a0, b0, a1, b1, ga0, gb0, ga1, gb1, ss0, ss1):
    wid = lax.axis_index("s") * NC + lax.axis_index("c")
    base = wid * TOK_W
    NCH = TOK_W // TCH
    p1s, p2s = (p1a, p1b), (p2a, p2b)
    avs, bvs = (a0, a1), (b0, b1)
    gsa, gsb, ssem = (ga0, ga1), (gb0, gb1), (ss0, ss1)

    def start(c, b):
        off = base + c * TCH
        pltpu.sync_copy(p1_hbm.at[pl.ds(off, TCH)], p1s[b])
        pltpu.sync_copy(p2_hbm.at[pl.ds(off, TCH)], p2s[b])
        return (pltpu.async_copy(y_hbm.at[p1s[b]], avs[b], gsa[b]),
                pltpu.async_copy(y_hbm.at[p2s[b]], bvs[b], gsb[b]))

    stores = [None, None]
    gn = start(0, 0)
    for c in range(NCH):
        b = c & 1
        gcur = gn
        if c + 1 < NCH:
            if stores[b ^ 1] is not None:
                stores[b ^ 1].wait()
                stores[b ^ 1] = None
            gn = start(c + 1, b ^ 1)
        gcur[0].wait()
        gcur[1].wait()
        a_v, b_v = avs[b], bvs[b]

        def add_row(r, _):
            for cc in range(H // 16):
                a_v[r, pl.ds(cc * 16, 16)] += b_v[r, pl.ds(cc * 16, 16)]
            return 0

        lax.fori_loop(0, TCH, add_row, 0)
        stores[b] = pltpu.async_copy(
            a_v, out_hbm.at[pl.ds(base + c * TCH, TCH)], ssem[b])
    for st in stores:
        if st is not None:
            st.wait()


def _gmm_body(eb_ref, x_ref, w_ref, wa_ref, bz_ref, y_ref):
    y = jnp.dot(x_ref[...].astype(jnp.bfloat16), w_ref[0],
                preferred_element_type=jnp.float32)
    y_ref[...] = y * wa_ref[...] + bz_ref[...]


def _gmm(eb, x_sorted, ws, wa_pad, bz_pad):
    grid_spec = pltpu.PrefetchScalarGridSpec(
        num_scalar_prefetch=1,
        grid=(G,),
        in_specs=[
            pl.BlockSpec((BLK_R, H), lambda t, eb: (t, 0)),
            pl.BlockSpec((1, H, H), lambda t, eb: (eb[t], 0, 0)),
            pl.BlockSpec((BLK_R, 1), lambda t, eb: (t, 0)),
            pl.BlockSpec((BLK_R, 1), lambda t, eb: (t, 0)),
        ],
        out_specs=pl.BlockSpec((BLK_R, H), lambda t, eb: (t, 0)),
    )
    return pl.pallas_call(
        _gmm_body,
        grid_spec=grid_spec,
        out_shape=jax.ShapeDtypeStruct((P, H), jnp.float32),
    )(eb, x_sorted, ws, wa_pad, bz_pad)


@jax.jit
def _moe(x, router_w, ws_bf16):
    i1, i2, w1, w2, bias, r1, r2, cnt = _router(x, router_w)
    i1, i2 = i1[:, 0], i2[:, 0]
    r1, r2 = r1[:, 0], r2[:, 0]
    counts = cnt.reshape(E).astype(jnp.int32)
    nblk = -(-counts // BLK_R)
    blk_off = jnp.concatenate([jnp.zeros(1, jnp.int32),
                               jnp.cumsum(nblk)[:-1].astype(jnp.int32)])
    cum_nblk = jnp.cumsum(nblk).astype(jnp.int32)
    t_iota = jnp.arange(G, dtype=jnp.int32)
    eb = jnp.minimum(jnp.searchsorted(cum_nblk, t_iota, side='right')
                     .astype(jnp.int32), E - 1)
    p1 = blk_off[i1] * BLK_R + r1
    p2 = blk_off[i2] * BLK_R + r2
    tok = jnp.arange(N, dtype=jnp.int32)
    spread = jnp.arange(P, dtype=jnp.int32) % N
    src_row = spread.at[p1].set(tok).at[p2].set(tok)
    wa_pad = (jnp.zeros(P, jnp.float32).at[p1].set(w1[:, 0])
              .at[p2].set(w2[:, 0]))[:, None]
    bz_pad = jnp.zeros(P, jnp.float32).at[p1].set(bias[:, 0])[:, None]
    x_sorted = _dispatch_gather(x, src_row)
    y = _gmm(eb, x_sorted, ws_bf16, wa_pad, bz_pad)
    out = _combine(y, p1, p2)
    return out


def kernel(hidden_states, router_w, expert_weights, expert_mapping):
    b, s, h = hidden_states.shape
    x = hidden_states.reshape(-1, h)
    out = _moe(x, router_w, expert_weights.astype(jnp.bfloat16))
    return out.reshape(b, s, h)


# speedup vs baseline: 1.7493x; 1.4148x over previous
import functools

import jax
import jax.numpy as jnp
from jax import lax
from jax.experimental import pallas as pl
from jax.experimental.pallas import tpu as pltpu
from jax.experimental.pallas import tpu_sc as plsc

H, E, K = 1024, 8, 2
N = 4096
BLK_R = 256
G = 40
P = G * BLK_R
RBLK = 256

NC, NS = 2, 16
NW = NC * NS
DROWS_W = P // NW
DCH = 16
TOK_W = N // NW
TCH = 16


def _router_body(x_ref, rw_ref, i1_ref, i2_ref, w1_ref, w2_ref, b_ref,
                 r1_ref, r2_ref, c_ref, base_ref):
    @pl.when(pl.program_id(0) == 0)
    def _init():
        base_ref[...] = jnp.zeros_like(base_ref)

    x = x_ref[...]
    logits = jnp.dot(x, rw_ref[...].T, preferred_element_type=jnp.float32)
    m = jnp.max(logits, axis=-1, keepdims=True)
    p = jnp.exp(logits - m)
    probs = p / jnp.sum(p, axis=-1, keepdims=True)
    iota = jax.lax.broadcasted_iota(jnp.int32, probs.shape, 1)
    m1 = jnp.max(probs, axis=-1, keepdims=True)
    i1 = jnp.min(jnp.where(probs == m1, iota, E), axis=-1, keepdims=True)
    probs2 = jnp.where(iota == i1, -jnp.inf, probs)
    m2 = jnp.max(probs2, axis=-1, keepdims=True)
    i2 = jnp.min(jnp.where(probs2 == m2, iota, E), axis=-1, keepdims=True)
    denom = m1 + m2 + 1e-6
    w1 = m1 / denom
    w2 = m2 / denom
    i1_ref[...] = i1
    i2_ref[...] = i2
    w1_ref[...] = w1
    w2_ref[...] = w2
    b_ref[...] = w1 * m1 + w2 * m2
    oh1 = (iota == i1).astype(jnp.float32)
    oh2 = (iota == i2).astype(jnp.float32)
    ri = jax.lax.broadcasted_iota(jnp.int32, (RBLK, RBLK), 0)
    ci = jax.lax.broadcasted_iota(jnp.int32, (RBLK, RBLK), 1)
    tri = (ri > ci).astype(jnp.float32)
    base = base_ref[...]
    r1 = base + jnp.dot(tri, oh1, preferred_element_type=jnp.float32)
    base = base + jnp.sum(oh1, axis=0, keepdims=True)
    r2 = base + jnp.dot(tri, oh2, preferred_element_type=jnp.float32)
    base = base + jnp.sum(oh2, axis=0, keepdims=True)
    base_ref[...] = base
    r1_ref[...] = jnp.sum(r1 * oh1, axis=-1, keepdims=True).astype(jnp.int32)
    r2_ref[...] = jnp.sum(r2 * oh2, axis=-1, keepdims=True).astype(jnp.int32)
    c_ref[...] = base


def _router(x, router_w):
    o = jax.ShapeDtypeStruct((N, 1), jnp.float32)
    oi = jax.ShapeDtypeStruct((N, 1), jnp.int32)
    spec = pl.BlockSpec((RBLK, 1), lambda i: (i, 0))
    return pl.pallas_call(
        _router_body,
        grid=(N // RBLK,),
        in_specs=[
            pl.BlockSpec((RBLK, H), lambda i: (i, 0)),
            pl.BlockSpec((E, H), lambda i: (0, 0)),
        ],
        out_specs=[spec, spec, spec, spec, spec, spec, spec,
                   pl.BlockSpec((1, E), lambda i: (0, 0))],
        out_shape=[oi, oi, o, o, o, oi, oi,
                   jax.ShapeDtypeStruct((1, E), jnp.float32)],
        scratch_shapes=[pltpu.VMEM((1, E), jnp.float32)],
    )(x, router_w)


_SC_MESH = plsc.VectorSubcoreMesh(core_axis_name="c", subcore_axis_name="s")


@functools.partial(
    pl.kernel,
    out_type=jax.ShapeDtypeStruct((P, H), jnp.float32),
    mesh=_SC_MESH,
    scratch_types=[
        pltpu.VMEM((TCH,), jnp.int32),
        pltpu.VMEM((TCH,), jnp.int32),
        pltpu.VMEM((TCH,), jnp.int32),
        pltpu.VMEM((TCH,), jnp.int32),
        pltpu.VMEM((TCH, H), jnp.float32),
        pltpu.VMEM((TCH, H), jnp.float32),
        pltpu.SemaphoreType.DMA,
        pltpu.SemaphoreType.DMA,
        pltpu.SemaphoreType.DMA,
        pltpu.SemaphoreType.DMA,
    ],
)
def _dispatch_scatter(x_hbm, p1_hbm, p2_hbm, out_hbm, i1a, i2a, i1b, i2b,
                      r0, r1, sa0, sb0, sa1, sb1):
    wid = lax.axis_index("s") * NC + lax.axis_index("c")
    base = wid * TOK_W
    NCH = TOK_W // TCH
    i1s, i2s = (i1a, i1b), (i2a, i2b)
    rows = (r0, r1)
    sas, sbs = (sa0, sa1), (sb0, sb1)
    scats = [None, None]
    for c in range(NCH):
        b = c & 1
        if scats[b] is not None:
            scats[b][0].wait()
            scats[b][1].wait()
            scats[b] = None
        off = base + c * TCH
        pltpu.sync_copy(x_hbm.at[pl.ds(off, TCH)], rows[b])
        pltpu.sync_copy(p1_hbm.at[pl.ds(off, TCH)], i1s[b])
        pltpu.sync_copy(p2_hbm.at[pl.ds(off, TCH)], i2s[b])
        scats[b] = (pltpu.async_copy(rows[b], out_hbm.at[i1s[b]], sas[b]),
                    pltpu.async_copy(rows[b], out_hbm.at[i2s[b]], sbs[b]))
    for sc in scats:
        if sc is not None:
            sc[0].wait()
            sc[1].wait()


@functools.partial(
    pl.kernel,
    out_type=(jax.ShapeDtypeStruct((N, H), jnp.float32),
              jax.ShapeDtypeStruct((N, H), jnp.float32)),
    mesh=_SC_MESH,
    scratch_types=[
        pltpu.VMEM((TCH,), jnp.int32),
        pltpu.VMEM((TCH,), jnp.int32),
        pltpu.VMEM((TCH,), jnp.int32),
        pltpu.VMEM((TCH,), jnp.int32),
        pltpu.VMEM((TCH, H), jnp.float32),
        pltpu.VMEM((TCH, H), jnp.float32),
        pltpu.VMEM((TCH, H), jnp.float32),
        pltpu.VMEM((TCH, H), jnp.float32),
        pltpu.SemaphoreType.DMA,
        pltpu.SemaphoreType.DMA,
        pltpu.SemaphoreType.DMA,
        pltpu.SemaphoreType.DMA,
        pltpu.SemaphoreType.DMA,
        pltpu.SemaphoreType.DMA,
        pltpu.SemaphoreType.DMA,
        pltpu.SemaphoreType.DMA,
    ],
)
def _combine(y_hbm, p1_hbm, p2_hbm, y1_hbm, y2_hbm, p1a, p2a, p1b, p2b,
             a0, b0, a1, b1, ga0, gb0, ga1, gb1, s10, s20, s11, s21):
    wid = lax.axis_index("s") * NC + lax.axis_index("c")
    base = wid * TOK_W
    NCH = TOK_W // TCH
    p1s, p2s = (p1a, p1b), (p2a, p2b)
    avs, bvs = (a0, a1), (b0, b1)
    gsa, gsb = (ga0, ga1), (gb0, gb1)
    s1s, s2s = (s10, s11), (s20, s21)

    def start(c, b):
        off = base + c * TCH
        pltpu.sync_copy(p1_hbm.at[pl.ds(off, TCH)], p1s[b])
        pltpu.sync_copy(p2_hbm.at[pl.ds(off, TCH)], p2s[b])
        return (pltpu.async_copy(y_hbm.at[p1s[b]], avs[b], gsa[b]),
                pltpu.async_copy(y_hbm.at[p2s[b]], bvs[b], gsb[b]))

    stores = [None, None]
    gn = start(0, 0)
    for c in range(NCH):
        b = c & 1
        gcur = gn
        if c + 1 < NCH:
            if stores[b ^ 1] is not None:
                stores[b ^ 1][0].wait()
                stores[b ^ 1][1].wait()
                stores[b ^ 1] = None
            gn = start(c + 1, b ^ 1)
        gcur[0].wait()
        gcur[1].wait()
        off = base + c * TCH
        stores[b] = (
            pltpu.async_copy(avs[b], y1_hbm.at[pl.ds(off, TCH)], s1s[b]),
            pltpu.async_copy(bvs[b], y2_hbm.at[pl.ds(off, TCH)], s2s[b]))
    for st in stores:
        if st is not None:
            st[0].wait()
            st[1].wait()


def _gmm_body(eb_ref, x_ref, w_ref, y_ref):
    y_ref[...] = jnp.dot(x_ref[...].astype(jnp.bfloat16), w_ref[0],
                         preferred_element_type=jnp.float32)


def _gmm(eb, x_sorted, ws):
    grid_spec = pltpu.PrefetchScalarGridSpec(
        num_scalar_prefetch=1,
        grid=(G,),
        in_specs=[
            pl.BlockSpec((BLK_R, H), lambda t, eb: (t, 0)),
            pl.BlockSpec((1, H, H), lambda t, eb: (eb[t], 0, 0)),
        ],
        out_specs=pl.BlockSpec((BLK_R, H), lambda t, eb: (t, 0)),
    )
    return pl.pallas_call(
        _gmm_body,
        grid_spec=grid_spec,
        out_shape=jax.ShapeDtypeStruct((P, H), jnp.float32),
    )(eb, x_sorted, ws)


def _final_body(y1_ref, y2_ref, w1_ref, w2_ref, b_ref, out_ref):
    out_ref[...] = (y1_ref[...] * w1_ref[...] + y2_ref[...] * w2_ref[...]
                    + b_ref[...])


def _final(y1, y2, w1, w2, bias):
    blk = 512
    wspec = pl.BlockSpec((blk, 1), lambda i: (i, 0))
    return pl.pallas_call(
        _final_body,
        grid=(N // blk,),
        in_specs=[
            pl.BlockSpec((blk, H), lambda i: (i, 0)),
            pl.BlockSpec((blk, H), lambda i: (i, 0)),
            wspec, wspec, wspec,
        ],
        out_specs=pl.BlockSpec((blk, H), lambda i: (i, 0)),
        out_shape=jax.ShapeDtypeStruct((N, H), jnp.float32),
    )(y1, y2, w1, w2, bias)


@jax.jit
def _moe(x, router_w, ws_bf16):
    i1, i2, w1, w2, bias, r1, r2, cnt = _router(x, router_w)
    i1, i2 = i1[:, 0], i2[:, 0]
    r1, r2 = r1[:, 0], r2[:, 0]
    counts = cnt.reshape(E).astype(jnp.int32)
    nblk = -(-counts // BLK_R)
    blk_off = jnp.concatenate([jnp.zeros(1, jnp.int32),
                               jnp.cumsum(nblk)[:-1].astype(jnp.int32)])
    cum_nblk = jnp.cumsum(nblk).astype(jnp.int32)
    t_iota = jnp.arange(G, dtype=jnp.int32)
    eb = jnp.minimum(
        jnp.sum((cum_nblk[None, :] <= t_iota[:, None]).astype(jnp.int32),
                axis=1), E - 1)
    p1 = blk_off[i1] * BLK_R + r1
    p2 = blk_off[i2] * BLK_R + r2
    x_sorted = _dispatch_scatter(x, p1, p2)
    y = _gmm(eb, x_sorted, ws_bf16)
    y1, y2 = _combine(y, p1, p2)
    return _final(y1, y2, w1, w2, bias)


def kernel(hidden_states, router_w, expert_weights, expert_mapping):
    b, s, h = hidden_states.shape
    x = hidden_states.reshape(-1, h)
    out = _moe(x, router_w, expert_weights.astype(jnp.bfloat16))
    return out.reshape(b, s, h)


# speedup vs baseline: 1.8875x; 1.0790x over previous
import functools

import jax
import jax.numpy as jnp
from jax import lax
from jax.experimental import pallas as pl
from jax.experimental.pallas import tpu as pltpu
from jax.experimental.pallas import tpu_sc as plsc

H, E, K = 1024, 8, 2
N = 4096
BLK_R = 512
G = 24
P = G * BLK_R
RBLK = 512

NC, NS = 2, 16
NW = NC * NS
TOK_W = N // NW
TCH = 16
DTCH = 32


def _router_body(x_ref, rw_ref, i1_ref, i2_ref, w1_ref, w2_ref, b_ref,
                 r1_ref, r2_ref, c_ref, base_ref):
    @pl.when(pl.program_id(0) == 0)
    def _init():
        base_ref[...] = jnp.zeros_like(base_ref)

    x = x_ref[...]
    logits = jnp.dot(x, rw_ref[...].T, preferred_element_type=jnp.float32)
    m = jnp.max(logits, axis=-1, keepdims=True)
    p = jnp.exp(logits - m)
    probs = p / jnp.sum(p, axis=-1, keepdims=True)
    iota = jax.lax.broadcasted_iota(jnp.int32, probs.shape, 1)
    m1 = jnp.max(probs, axis=-1, keepdims=True)
    i1 = jnp.min(jnp.where(probs == m1, iota, E), axis=-1, keepdims=True)
    probs2 = jnp.where(iota == i1, -jnp.inf, probs)
    m2 = jnp.max(probs2, axis=-1, keepdims=True)
    i2 = jnp.min(jnp.where(probs2 == m2, iota, E), axis=-1, keepdims=True)
    denom = m1 + m2 + 1e-6
    w1 = m1 / denom
    w2 = m2 / denom
    i1_ref[...] = i1
    i2_ref[...] = i2
    w1_ref[...] = w1
    w2_ref[...] = w2
    b_ref[...] = w1 * m1 + w2 * m2
    oh1 = (iota == i1).astype(jnp.float32)
    oh2 = (iota == i2).astype(jnp.float32)
    ri = jax.lax.broadcasted_iota(jnp.int32, (RBLK, RBLK), 0)
    ci = jax.lax.broadcasted_iota(jnp.int32, (RBLK, RBLK), 1)
    tri = (ri > ci).astype(jnp.float32)
    base = base_ref[...]
    r1 = base + jnp.dot(tri, oh1, preferred_element_type=jnp.float32)
    base = base + jnp.sum(oh1, axis=0, keepdims=True)
    r2 = base + jnp.dot(tri, oh2, preferred_element_type=jnp.float32)
    base = base + jnp.sum(oh2, axis=0, keepdims=True)
    base_ref[...] = base
    r1_ref[...] = jnp.sum(r1 * oh1, axis=-1, keepdims=True).astype(jnp.int32)
    r2_ref[...] = jnp.sum(r2 * oh2, axis=-1, keepdims=True).astype(jnp.int32)
    c_ref[...] = base


def _router(x, router_w):
    o = jax.ShapeDtypeStruct((N, 1), jnp.float32)
    oi = jax.ShapeDtypeStruct((N, 1), jnp.int32)
    spec = pl.BlockSpec((RBLK, 1), lambda i: (i, 0))
    return pl.pallas_call(
        _router_body,
        grid=(N // RBLK,),
        in_specs=[
            pl.BlockSpec((RBLK, H), lambda i: (i, 0)),
            pl.BlockSpec((E, H), lambda i: (0, 0)),
        ],
        out_specs=[spec, spec, spec, spec, spec, spec, spec,
                   pl.BlockSpec((1, E), lambda i: (0, 0))],
        out_shape=[oi, oi, o, o, o, oi, oi,
                   jax.ShapeDtypeStruct((1, E), jnp.float32)],
        scratch_shapes=[pltpu.VMEM((1, E), jnp.float32)],
    )(x, router_w)


_SC_MESH = plsc.VectorSubcoreMesh(core_axis_name="c", subcore_axis_name="s")


@functools.partial(
    pl.kernel,
    out_type=jax.ShapeDtypeStruct((P, H), jnp.float32),
    mesh=_SC_MESH,
    scratch_types=[
        pltpu.VMEM((DTCH,), jnp.int32),
        pltpu.VMEM((DTCH,), jnp.int32),
        pltpu.VMEM((DTCH,), jnp.int32),
        pltpu.VMEM((DTCH,), jnp.int32),
        pltpu.VMEM((DTCH, H), jnp.float32),
        pltpu.VMEM((DTCH, H), jnp.float32),
        pltpu.SemaphoreType.DMA,
        pltpu.SemaphoreType.DMA,
        pltpu.SemaphoreType.DMA,
        pltpu.SemaphoreType.DMA,
    ],
)
def _dispatch_scatter(x_hbm, p1_hbm, p2_hbm, out_hbm, i1a, i2a, i1b, i2b,
                      r0, r1, sa0, sb0, sa1, sb1):
    wid = lax.axis_index("s") * NC + lax.axis_index("c")
    base = wid * TOK_W
    NCH = TOK_W // DTCH
    i1s, i2s = (i1a, i1b), (i2a, i2b)
    rows = (r0, r1)
    sas, sbs = (sa0, sa1), (sb0, sb1)
    scats = [None, None]
    for c in range(NCH):
        b = c & 1
        if scats[b] is not None:
            scats[b][0].wait()
            scats[b][1].wait()
            scats[b] = None
        off = base + c * DTCH
        pltpu.sync_copy(x_hbm.at[pl.ds(off, DTCH)], rows[b])
        pltpu.sync_copy(p1_hbm.at[pl.ds(off, DTCH)], i1s[b])
        pltpu.sync_copy(p2_hbm.at[pl.ds(off, DTCH)], i2s[b])
        scats[b] = (pltpu.async_copy(rows[b], out_hbm.at[i1s[b]], sas[b]),
                    pltpu.async_copy(rows[b], out_hbm.at[i2s[b]], sbs[b]))
    for sc in scats:
        if sc is not None:
            sc[0].wait()
            sc[1].wait()


@functools.partial(
    pl.kernel,
    out_type=(jax.ShapeDtypeStruct((N, H), jnp.float32),
              jax.ShapeDtypeStruct((N, H), jnp.float32)),
    mesh=_SC_MESH,
    scratch_types=[
        pltpu.VMEM((TCH,), jnp.int32),
        pltpu.VMEM((TCH,), jnp.int32),
        pltpu.VMEM((TCH,), jnp.int32),
        pltpu.VMEM((TCH,), jnp.int32),
        pltpu.VMEM((TCH, H), jnp.float32),
        pltpu.VMEM((TCH, H), jnp.float32),
        pltpu.VMEM((TCH, H), jnp.float32),
        pltpu.VMEM((TCH, H), jnp.float32),
        pltpu.SemaphoreType.DMA,
        pltpu.SemaphoreType.DMA,
        pltpu.SemaphoreType.DMA,
        pltpu.SemaphoreType.DMA,
        pltpu.SemaphoreType.DMA,
        pltpu.SemaphoreType.DMA,
        pltpu.SemaphoreType.DMA,
        pltpu.SemaphoreType.DMA,
    ],
)
def _combine(y_hbm, p1_hbm, p2_hbm, y1_hbm, y2_hbm, p1a, p2a, p1b, p2b,
             a0, b0, a1, b1, ga0, gb0, ga1, gb1, s10, s20, s11, s21):
    wid = lax.axis_index("s") * NC + lax.axis_index("c")
    base = wid * TOK_W
    NCH = TOK_W // TCH
    p1s, p2s = (p1a, p1b), (p2a, p2b)
    avs, bvs = (a0, a1), (b0, b1)
    gsa, gsb = (ga0, ga1), (gb0, gb1)
    s1s, s2s = (s10, s11), (s20, s21)

    def start(c, b):
        off = base + c * TCH
        pltpu.sync_copy(p1_hbm.at[pl.ds(off, TCH)], p1s[b])
        pltpu.sync_copy(p2_hbm.at[pl.ds(off, TCH)], p2s[b])
        return (pltpu.async_copy(y_hbm.at[p1s[b]], avs[b], gsa[b]),
                pltpu.async_copy(y_hbm.at[p2s[b]], bvs[b], gsb[b]))

    stores = [None, None]
    gn = start(0, 0)
    for c in range(NCH):
        b = c & 1
        gcur = gn
        if c + 1 < NCH:
            if stores[b ^ 1] is not None:
                stores[b ^ 1][0].wait()
                stores[b ^ 1][1].wait()
                stores[b ^ 1] = None
            gn = start(c + 1, b ^ 1)
        gcur[0].wait()
        gcur[1].wait()
        off = base + c * TCH
        stores[b] = (
            pltpu.async_copy(avs[b], y1_hbm.at[pl.ds(off, TCH)], s1s[b]),
            pltpu.async_copy(bvs[b], y2_hbm.at[pl.ds(off, TCH)], s2s[b]))
    for st in stores:
        if st is not None:
            st[0].wait()
            st[1].wait()


def _gmm_body(eb_ref, x_ref, w_ref, y_ref):
    y_ref[...] = jnp.dot(x_ref[...].astype(jnp.bfloat16), w_ref[0],
                         preferred_element_type=jnp.float32)


def _gmm(eb, x_sorted, ws):
    grid_spec = pltpu.PrefetchScalarGridSpec(
        num_scalar_prefetch=1,
        grid=(G,),
        in_specs=[
            pl.BlockSpec((BLK_R, H), lambda t, eb: (t, 0)),
            pl.BlockSpec((1, H, H), lambda t, eb: (eb[t], 0, 0)),
        ],
        out_specs=pl.BlockSpec((BLK_R, H), lambda t, eb: (t, 0)),
    )
    return pl.pallas_call(
        _gmm_body,
        grid_spec=grid_spec,
        out_shape=jax.ShapeDtypeStruct((P, H), jnp.float32),
    )(eb, x_sorted, ws)


def _final_body(y1_ref, y2_ref, w1_ref, w2_ref, b_ref, out_ref):
    out_ref[...] = (y1_ref[...] * w1_ref[...] + y2_ref[...] * w2_ref[...]
                    + b_ref[...])


def _final(y1, y2, w1, w2, bias):
    blk = 512
    wspec = pl.BlockSpec((blk, 1), lambda i: (i, 0))
    return pl.pallas_call(
        _final_body,
        grid=(N // blk,),
        in_specs=[
            pl.BlockSpec((blk, H), lambda i: (i, 0)),
            pl.BlockSpec((blk, H), lambda i: (i, 0)),
            wspec, wspec, wspec,
        ],
        out_specs=pl.BlockSpec((blk, H), lambda i: (i, 0)),
        out_shape=jax.ShapeDtypeStruct((N, H), jnp.float32),
    )(y1, y2, w1, w2, bias)


@jax.jit
def _moe(x, router_w, ws_bf16):
    i1, i2, w1, w2, bias, r1, r2, cnt = _router(x, router_w)
    i1, i2 = i1[:, 0], i2[:, 0]
    r1, r2 = r1[:, 0], r2[:, 0]
    counts = cnt.reshape(E).astype(jnp.int32)
    nblk = -(-counts // BLK_R)
    blk_off = jnp.concatenate([jnp.zeros(1, jnp.int32),
                               jnp.cumsum(nblk)[:-1].astype(jnp.int32)])
    cum_nblk = jnp.cumsum(nblk).astype(jnp.int32)
    t_iota = jnp.arange(G, dtype=jnp.int32)
    eb = jnp.minimum(
        jnp.sum((cum_nblk[None, :] <= t_iota[:, None]).astype(jnp.int32),
                axis=1), E - 1)
    p1 = blk_off[i1] * BLK_R + r1
    p2 = blk_off[i2] * BLK_R + r2
    x_sorted = _dispatch_scatter(x, p1, p2)
    y = _gmm(eb, x_sorted, ws_bf16)
    y1, y2 = _combine(y, p1, p2)
    return _final(y1, y2, w1, w2, bias)


def kernel(hidden_states, router_w, expert_weights, expert_mapping):
    b, s, h = hidden_states.shape
    x = hidden_states.reshape(-1, h)
    out = _moe(x, router_w, expert_weights.astype(jnp.bfloat16))
    return out.reshape(b, s, h)
